# baseline (speedup 1.0000x reference)
import functools

import jax
import jax.numpy as jnp
from jax import lax
from jax.experimental import pallas as pl
from jax.experimental.pallas import tpu as pltpu
from jax.experimental.pallas import tpu_sc as plsc

NC = 2
NS = 16
NW = NC * NS
L = 16

CHUNK = 80
GROUPS = CHUNK // L
FLUSH = 25
STRIPE = 1000


def _tec_body(D, per_w, n_nodes, table_hbm, src_hbm, tgt_hbm, out_hbm,
              table_sh, sidx0, tidx0, sidx1, tidx1,
              srows0, trows0, srows1, trows1,
              acc_v, out_v, sem0, sem1, isem0, isem1):
  wid = lax.axis_index("s") * NC + lax.axis_index("c")
  sid = lax.axis_index("s")
  n_chunks = per_w // CHUNK
  base = wid * per_w
  nslice = D // L
  bufs = ((sidx0, tidx0, srows0, trows0, sem0, isem0),
          (sidx1, tidx1, srows1, trows1, sem1, isem1))

  @pl.when(sid < n_nodes // STRIPE)
  def _():
    off = pl.multiple_of(sid * STRIPE, 8)
    pltpu.sync_copy(table_hbm.at[pl.ds(off, STRIPE)], table_sh.at[pl.ds(off, STRIPE)])

  plsc.subcore_barrier()

  def idx_refs(ci):
    off = pl.multiple_of(base + ci * CHUNK, 8)
    return src_hbm.at[pl.ds(off, CHUNK)], tgt_hbm.at[pl.ds(off, CHUNK)]

  def fire_idx(ci, b):
    sidx, tidx, _, _, _, isem = bufs[b]
    shbm, thbm = idx_refs(ci)
    pltpu.async_copy(shbm, sidx, isem)
    pltpu.async_copy(thbm, tidx, isem)

  def wait_idx(ci, b):
    sidx, tidx, _, _, _, isem = bufs[b]
    shbm, thbm = idx_refs(ci)
    pltpu.make_async_copy(shbm, sidx, isem).wait()
    pltpu.make_async_copy(thbm, tidx, isem).wait()

  def fire_gather(b):
    sidx, tidx, srows, trows, sem, _ = bufs[b]
    pltpu.async_copy(table_sh.at[sidx], srows, sem)
    pltpu.async_copy(table_sh.at[tidx], trows, sem)

  def wait_gather(b):
    sidx, tidx, srows, trows, sem, _ = bufs[b]
    pltpu.make_async_copy(table_sh.at[sidx], srows, sem).wait()
    pltpu.make_async_copy(table_sh.at[tidx], trows, sem).wait()

  def compute(ci, b):
    _, _, srows, trows, _, _ = bufs[b]
    slot = lax.rem(ci, FLUSH)

    def group_body(g, c2):
      eb = g * L
      for j in range(L):
        e = eb + j
        acc = srows[e, pl.ds(0, L)] * trows[e, pl.ds(0, L)]
        for k in range(1, nslice):
          acc = acc + srows[e, pl.ds(k * L, L)] * trows[e, pl.ds(k * L, L)]
        acc_v[pl.ds(j * L, L)] = acc
      rows = lax.iota(jnp.int32, L) * L
      tot = plsc.load_gather(acc_v, [rows])
      for i in range(1, L):
        tot = tot + plsc.load_gather(acc_v, [rows + i])
      out_v[pl.ds(slot * CHUNK + eb, L)] = 1.0 / (1.0 + jnp.exp(-tot))
      return c2

    lax.fori_loop(0, GROUPS, group_body, 0)

  s0hbm, t0hbm = idx_refs(0)
  pltpu.sync_copy(s0hbm, sidx0)
  pltpu.sync_copy(t0hbm, tidx0)
  fire_gather(0)
  fire_idx(1, 1)

  def outer(io, carry):
    for b in range(2):
      ci = io * 2 + b
      ob = 1 - b

      @pl.when(ci + 1 < n_chunks)
      def _():
        wait_idx(ci + 1, ob)
        fire_gather(ob)

      wait_gather(b)
      compute(ci, b)

      @pl.when(ci + 2 < n_chunks)
      def _():
        fire_idx(ci + 2, b)

      @pl.when(lax.rem(ci, FLUSH) == FLUSH - 1)
      def _():
        foff = pl.multiple_of(base + (ci - (FLUSH - 1)) * CHUNK, 8)
        pltpu.sync_copy(out_v, out_hbm.at[pl.ds(foff, FLUSH * CHUNK)])

    return carry

  lax.fori_loop(0, n_chunks // 2, outer, 0)


def _link_predict(table, src, tgt):
  E = src.shape[0]
  n_nodes, D = table.shape
  assert E % NW == 0
  per_w = E // NW
  n_chunks = per_w // CHUNK
  assert per_w % CHUNK == 0 and D % L == 0
  assert n_chunks % 2 == 0 and n_chunks % FLUSH == 0
  assert n_nodes % STRIPE == 0 and n_nodes // STRIPE <= NS

  mesh = plsc.VectorSubcoreMesh(core_axis_name="c", subcore_axis_name="s")
  k = pl.kernel(
      functools.partial(_tec_body, D, per_w, n_nodes),
      out_type=jax.ShapeDtypeStruct((E,), jnp.float32),
      mesh=mesh,
      compiler_params=pltpu.CompilerParams(needs_layout_passes=False),
      scratch_types=[
          pltpu.VMEM_SHARED((n_nodes, D), jnp.float32),
          pltpu.VMEM((CHUNK,), jnp.int32),
          pltpu.VMEM((CHUNK,), jnp.int32),
          pltpu.VMEM((CHUNK,), jnp.int32),
          pltpu.VMEM((CHUNK,), jnp.int32),
          pltpu.VMEM((CHUNK, D), jnp.float32),
          pltpu.VMEM((CHUNK, D), jnp.float32),
          pltpu.VMEM((CHUNK, D), jnp.float32),
          pltpu.VMEM((CHUNK, D), jnp.float32),
          pltpu.VMEM((L * L,), jnp.float32),
          pltpu.VMEM((FLUSH * CHUNK,), jnp.float32),
          pltpu.SemaphoreType.DMA,
          pltpu.SemaphoreType.DMA,
          pltpu.SemaphoreType.DMA,
          pltpu.SemaphoreType.DMA,
      ],
  )
  return k(table, src, tgt)


def kernel(node_embedding_matrix, pos_edge_index, neg_edge_index, batch_train_x_index):
  src = jnp.concatenate([pos_edge_index[0], neg_edge_index[0]]).astype(jnp.int32)
  tgt = jnp.concatenate([pos_edge_index[1], neg_edge_index[1]]).astype(jnp.int32)
  return _link_predict(node_embedding_matrix, src, tgt)


# speedup vs baseline: 1.2399x; 1.2399x over previous
import functools

import jax
import jax.numpy as jnp
from jax import lax
from jax.experimental import pallas as pl
from jax.experimental.pallas import tpu as pltpu
from jax.experimental.pallas import tpu_sc as plsc

NC = 2
NS = 16
NW = NC * NS
L = 16

CHUNK = 80
GROUPS = CHUNK // L
NSLOT = 3


def _norms_tc_body(table_ref, out_ref):
  x = table_ref[...]
  out_ref[...] = jnp.sum(x * x, axis=1)


def _node_norms(table):
  n_nodes, _ = table.shape
  return pl.pallas_call(
      _norms_tc_body,
      out_shape=jax.ShapeDtypeStruct((n_nodes,), jnp.float32),
  )(table)


def _tec_body(D, per_w, n_nodes, table_hbm, src_hbm, tgt_hbm, norms_hbm, out_hbm,
              sidx_all, tidx_all, norms_v, rows0, rows1, rows2,
              acc_v, out_v, psem0, psem1, psem2, asem0, asem1, asem2):
  wid = lax.axis_index("s") * NC + lax.axis_index("c")
  n_chunks = per_w // CHUNK
  base = wid * per_w
  nslice = D // L
  rows = (rows0, rows1, rows2)
  psems = (psem0, psem1, psem2)
  asems = (asem0, asem1, asem2)

  pltpu.sync_copy(src_hbm.at[pl.ds(base, per_w)], sidx_all)
  pltpu.sync_copy(tgt_hbm.at[pl.ds(base, per_w)], tidx_all)
  pltpu.sync_copy(norms_hbm, norms_v)

  def sidx_ref(ci):
    off = pl.multiple_of(ci * CHUNK, 8)
    return sidx_all.at[pl.ds(off, CHUNK)]

  def tidx_ref(ci):
    off = pl.multiple_of(ci * CHUNK, 8)
    return tidx_all.at[pl.ds(off, CHUNK)]

  def fire_plain(ci, s):
    pltpu.async_copy(table_hbm.at[sidx_ref(ci)], rows[s], psems[s])

  def wait_plain(ci, s):
    pltpu.make_async_copy(table_hbm.at[sidx_ref(ci)], rows[s], psems[s]).wait()

  def fire_add(ci, s):
    pltpu.async_copy(table_hbm.at[tidx_ref(ci)], rows[s], asems[s], add=True)

  def wait_add(ci, s):
    pltpu.make_async_copy(table_hbm.at[tidx_ref(ci)], rows[s], asems[s]).wait()

  def compute(ci, s):
    r = rows[s]

    def group_body(g, c2):
      eb = g * L
      for j in range(L):
        e = eb + j
        v = r[e, pl.ds(0, L)]
        acc = v * v
        for k in range(1, nslice):
          v = r[e, pl.ds(k * L, L)]
          acc = acc + v * v
        acc_v[pl.ds(j * L, L)] = acc
      lanes = lax.iota(jnp.int32, L) * L
      ss = plsc.load_gather(acc_v, [lanes])
      for i in range(1, L):
        ss = ss + plsc.load_gather(acc_v, [lanes + i])
      eoff = ci * CHUNK + eb
      ns = plsc.load_gather(norms_v, [sidx_all[pl.ds(eoff, L)]])
      nt = plsc.load_gather(norms_v, [tidx_all[pl.ds(eoff, L)]])
      tot = 0.5 * (ss - ns - nt)
      out_v[pl.ds(eoff, L)] = 1.0 / (1.0 + jnp.exp(-tot))
      return c2

    lax.fori_loop(0, GROUPS, group_body, 0)

  fire_plain(0, 0)
  fire_plain(1, 1)
  wait_plain(0, 0)
  fire_add(0, 0)

  def outer(io, carry):
    for b in range(NSLOT):
      ci = io * NSLOT + b

      @pl.when(ci < n_chunks)
      def _():
        @pl.when(ci + 1 < n_chunks)
        def _():
          wait_plain(ci + 1, (b + 1) % NSLOT)
          fire_add(ci + 1, (b + 1) % NSLOT)

        @pl.when(ci + 2 < n_chunks)
        def _():
          fire_plain(ci + 2, (b + 2) % NSLOT)

        wait_add(ci, b)
        compute(ci, b)

    return carry

  lax.fori_loop(0, (n_chunks + NSLOT - 1) // NSLOT, outer, 0)
  pltpu.sync_copy(out_v, out_hbm.at[pl.ds(wid * per_w, per_w)])


def _link_predict(table, src, tgt):
  E = src.shape[0]
  n_nodes, D = table.shape
  assert E % NW == 0
  per_w = E // NW
  assert per_w % CHUNK == 0 and D % L == 0

  norms = _node_norms(table)
  mesh = plsc.VectorSubcoreMesh(core_axis_name="c", subcore_axis_name="s")
  k = pl.kernel(
      functools.partial(_tec_body, D, per_w, n_nodes),
      out_type=jax.ShapeDtypeStruct((E,), jnp.float32),
      mesh=mesh,
      compiler_params=pltpu.CompilerParams(needs_layout_passes=False),
      scratch_types=[
          pltpu.VMEM((per_w,), jnp.int32),
          pltpu.VMEM((per_w,), jnp.int32),
          pltpu.VMEM((n_nodes,), jnp.float32),
          pltpu.VMEM((CHUNK, D), jnp.float32),
          pltpu.VMEM((CHUNK, D), jnp.float32),
          pltpu.VMEM((CHUNK, D), jnp.float32),
          pltpu.VMEM((L * L,), jnp.float32),
          pltpu.VMEM((per_w,), jnp.float32),
          pltpu.SemaphoreType.DMA,
          pltpu.SemaphoreType.DMA,
          pltpu.SemaphoreType.DMA,
          pltpu.SemaphoreType.DMA,
          pltpu.SemaphoreType.DMA,
          pltpu.SemaphoreType.DMA,
      ],
  )
  return k(table, src, tgt, norms)


def kernel(node_embedding_matrix, pos_edge_index, neg_edge_index, batch_train_x_index):
  src = jnp.concatenate([pos_edge_index[0], neg_edge_index[0]]).astype(jnp.int32)
  tgt = jnp.concatenate([pos_edge_index[1], neg_edge_index[1]]).astype(jnp.int32)
  return _link_predict(node_embedding_matrix, src, tgt)
